# baseline (speedup 1.0000x reference)
import functools

import jax
import jax.numpy as jnp
from jax import lax
from jax.experimental import pallas as pl
from jax.experimental.pallas import tpu as pltpu
from jax.experimental.pallas import tpu_sc as plsc

N, E, F_IN = 10000, 320000, 128
HEADS, NHID, NCLASS = 8, 8, 40
HC = HEADS * NHID
ROW1 = 72
ROW2 = 48
NC, NS = 2, 16
NW = NC * NS
B = 272
NCHUNK = 38
EPW = B * NCHUNK
NPAD = 10240
RPT = NPAD // NS

_f32 = jnp.float32


def _ea_sum_body(ea_ref, out_ref):
    out_ref[...] = jnp.sum(ea_ref[...]).reshape(1, 1) * (1.0 / E)


def _ea_mean(ea):
    return pl.pallas_call(
        _ea_sum_body,
        out_shape=jax.ShapeDtypeStruct((1, 1), _f32),
        in_specs=[pl.BlockSpec((2500, 128), lambda: (0, 0))],
        out_specs=pl.BlockSpec((1, 1), lambda: (0, 0)),
    )(ea.reshape(2500, 128))


def _prep1_body(x_ref, w_ref, asrc_ref, adst_ref, tab_ref, ad_ref):
    xs = jnp.dot(x_ref[...], w_ref[...], preferred_element_type=_f32)
    tab_ref[:, :HC] = xs
    tab_ref[:, HC:] = jnp.dot(xs, asrc_ref[...], preferred_element_type=_f32)
    ad_ref[...] = jnp.dot(xs, adst_ref[...], preferred_element_type=_f32)


def _prep1(x, W1, Asrc, Adst):
    R = 1000
    return pl.pallas_call(
        _prep1_body,
        grid=(N // R,),
        out_shape=[jax.ShapeDtypeStruct((N, ROW1), _f32),
                   jax.ShapeDtypeStruct((N, HEADS), _f32)],
        in_specs=[pl.BlockSpec((R, F_IN), lambda i: (i, 0)),
                  pl.BlockSpec((F_IN, HC), lambda i: (0, 0)),
                  pl.BlockSpec((HC, HEADS), lambda i: (0, 0)),
                  pl.BlockSpec((HC, HEADS), lambda i: (0, 0))],
        out_specs=[pl.BlockSpec((R, ROW1), lambda i: (i, 0)),
                   pl.BlockSpec((R, HEADS), lambda i: (i, 0))],
    )(x, W1, Asrc, Adst)


def _mid_body(tab_ref, accA_ref, accB_ref,
              ad_ref, eam_ref, c1_ref, b1_ref, exp8_ref, w2_ref, a2s_ref,
              a2d_ref, tab2_ref, as2_ref, ad2_ref):
    xs = tab_ref[:, :HC]
    al = tab_ref[:, HC:] + ad_ref[...] + eam_ref[0, 0] * c1_ref[...]
    ex = jnp.exp(jnp.maximum(al, 0.2 * al))
    exp8 = exp8_ref[...]
    num = accA_ref[0, :, :HC] + accB_ref[0, :, :HC] \
        + jnp.dot(ex, exp8, preferred_element_type=_f32) * xs
    den = accA_ref[0, :, HC:] + accB_ref[0, :, HC:] + ex
    h = num / jnp.dot(den, exp8, preferred_element_type=_f32) + b1_ref[...]
    h = jnp.where(h > 0, h, jnp.exp(jnp.minimum(h, 0.0)) - 1.0)
    xs2 = jnp.dot(h, w2_ref[...], preferred_element_type=_f32)
    tab2_ref[:, :NCLASS] = xs2
    tab2_ref[:, NCLASS:] = jnp.zeros_like(tab2_ref[:, NCLASS:])
    as2_ref[...] = jnp.dot(xs2, a2s_ref[...], preferred_element_type=_f32)
    ad2_ref[...] = jnp.dot(xs2, a2d_ref[...], preferred_element_type=_f32)


def _mid(tab, acc, ad, eam, c1, b1, exp8, W2, a2s, a2d):
    R = 1000
    return pl.pallas_call(
        _mid_body,
        grid=(N // R,),
        out_shape=[jax.ShapeDtypeStruct((N, ROW2), _f32),
                   jax.ShapeDtypeStruct((N, 1), _f32),
                   jax.ShapeDtypeStruct((N, 1), _f32)],
        in_specs=[pl.BlockSpec((R, ROW1), lambda i: (i, 0)),
                  pl.BlockSpec((1, R, ROW1), lambda i: (0, i, 0)),
                  pl.BlockSpec((1, R, ROW1), lambda i: (1, i, 0)),
                  pl.BlockSpec((R, HEADS), lambda i: (i, 0)),
                  pl.BlockSpec((1, 1), lambda i: (0, 0)),
                  pl.BlockSpec((1, HEADS), lambda i: (0, 0)),
                  pl.BlockSpec((1, HC), lambda i: (0, 0)),
                  pl.BlockSpec((HEADS, HC), lambda i: (0, 0)),
                  pl.BlockSpec((HC, NCLASS), lambda i: (0, 0)),
                  pl.BlockSpec((NCLASS, 1), lambda i: (0, 0)),
                  pl.BlockSpec((NCLASS, 1), lambda i: (0, 0))],
        out_specs=[pl.BlockSpec((R, ROW2), lambda i: (i, 0)),
                   pl.BlockSpec((R, 1), lambda i: (i, 0)),
                   pl.BlockSpec((R, 1), lambda i: (i, 0))],
    )(tab, acc, acc, ad, eam, c1, b1, exp8, W2, a2s, a2d)


def _final_body(tab2_ref, as2_ref, accA_ref, accB_ref,
                ad2_ref, eam_ref, c2_ref, b2_ref, out_ref):
    xs2 = tab2_ref[:, :NCLASS]
    al = as2_ref[...] + ad2_ref[...] + eam_ref[0, 0] * c2_ref[0, 0]
    ex = jnp.exp(jnp.maximum(al, 0.2 * al))
    num = accA_ref[0, :, :NCLASS] + accB_ref[0, :, :NCLASS] + ex * xs2
    den = accA_ref[0, :, NCLASS:NCLASS + 1] \
        + accB_ref[0, :, NCLASS:NCLASS + 1] + ex
    o = num / den + b2_ref[...]
    m = jnp.max(o, axis=1, keepdims=True)
    s = jnp.sum(jnp.exp(o - m), axis=1, keepdims=True)
    out_ref[...] = o - m - jnp.log(s)


def _final(tab2, as2, acc, ad2, eam, c2, b2):
    R = 1000
    return pl.pallas_call(
        _final_body,
        grid=(N // R,),
        out_shape=jax.ShapeDtypeStruct((N, NCLASS), _f32),
        in_specs=[pl.BlockSpec((R, ROW2), lambda i: (i, 0)),
                  pl.BlockSpec((R, 1), lambda i: (i, 0)),
                  pl.BlockSpec((1, R, ROW2), lambda i: (0, i, 0)),
                  pl.BlockSpec((1, R, ROW2), lambda i: (1, i, 0)),
                  pl.BlockSpec((R, 1), lambda i: (i, 0)),
                  pl.BlockSpec((1, 1), lambda i: (0, 0)),
                  pl.BlockSpec((1, 1), lambda i: (0, 0)),
                  pl.BlockSpec((1, NCLASS), lambda i: (0, 0))],
        out_specs=pl.BlockSpec((R, NCLASS), lambda i: (i, 0)),
    )(tab2, as2, acc, acc, ad2, eam, c2, b2)


def _pipeline(issue_idx, wait_idx, issue_gather, wait_gather, compute,
              issue_scatter, wait_scatter):
    issue_idx(0, 0)
    issue_idx(1, 1)
    wait_idx(0, 0)
    issue_gather(0, 0, 0)
    for j in (0, 1):
        wait_idx(j + 1, j + 1)
        issue_gather((j + 1) % 3, j + 1, (j + 1) % 2)
        issue_idx(j + 2, j + 2)
        wait_gather(j % 3, j % 4, j % 2)
        compute(j % 3, j % 4, j % 2)
        issue_scatter(j % 3, j % 4, j % 2)

    def superstep(ss, _):
        for i in range(12):
            j = 12 * ss + 2 + i
            r, q, p = (2 + i) % 3, (2 + i) % 4, i % 2
            wait_scatter(i % 3, i % 4, p)

            def _prefetch(r1=(3 + i) % 3, q1=(3 + i) % 4, p1=(i + 1) % 2,
                          jj=j + 1):
                wait_idx(jj, q1)
                issue_gather(r1, q1, p1)

            def _nextidx(jj=j + 2, q2=i % 4):
                issue_idx(jj, q2)

            pl.when(j + 1 < NCHUNK)(_prefetch)
            pl.when(j + 2 < NCHUNK)(_nextidx)
            wait_gather(r, q, p)
            compute(r, q, p)
            issue_scatter(r, q, p)
        return 0

    lax.fori_loop(0, (NCHUNK - 2) // 12, superstep, 0)
    wait_scatter(0, 0, 0)
    wait_scatter(1, 1, 1)


def _vgather(v, idx):
    return lax.gather(
        v, idx.reshape(16, 1),
        lax.GatherDimensionNumbers(offset_dims=(), collapsed_slice_dims=(0,),
                                   start_index_map=(0,)),
        slice_sizes=(1,), mode=lax.GatherScatterMode.PROMISE_IN_BOUNDS)


_MESH = plsc.VectorSubcoreMesh(core_axis_name="c", subcore_axis_name="s",
                               num_cores=NC, num_subcores=NS)
_SC_PARAMS = pltpu.CompilerParams(needs_layout_passes=False,
                                  use_tc_tiling_on_sc=False)
_IDX3 = [pltpu.VMEM((B,), jnp.int32), pltpu.VMEM((B,), jnp.int32),
         pltpu.VMEM((B,), _f32)]
_SEM9 = [pltpu.SemaphoreType.DMA] * 9


def _edges1_body(tab, adt, src, dst, ea, c16, zm, outm,
                 accm, rows0, rows1, rows2, adg0, adg1, exb, c16b,
                 sb0, db0, eb0, sb1, db1, eb1, sb2, db2, eb2, sb3, db3, eb3,
                 sg0, sg1, sg2, si0, si1, si2, si3, ss0, ss1):
    rows = [rows0, rows1, rows2]
    adg = [adg0, adg1]
    srcb, dstb = [sb0, sb1, sb2, sb3], [db0, db1, db2, db3]
    eab = [eb0, eb1, eb2, eb3]
    semg, semi, sems = [sg0, sg1, sg2], [si0, si1, si2, si3], [ss0, ss1]

    c = lax.axis_index("c")
    s = lax.axis_index("s")
    wid = c * NS + s
    pltpu.sync_copy(zm, accm.at[pl.ds(s * RPT, RPT)])
    pltpu.sync_copy(c16, c16b)
    plsc.subcore_barrier()

    iota = lax.iota(jnp.int32, 16)
    lane8 = jnp.bitwise_and(iota, 7)
    pair8 = jnp.right_shift(iota, 3)
    c16v = c16b[...]
    acols = HC + lane8
    arow_c = [2 * i + pair8 for i in range(8)]
    bidx_e0 = [2 * k + pair8 for k in range(4)]
    bidx_e1 = [8 + 2 * k + pair8 for k in range(4)]

    def issue_idx(j, q):
        base = wid * EPW + j * B
        pltpu.async_copy(src.at[pl.ds(base, B)], srcb[q], semi[q])
        pltpu.async_copy(dst.at[pl.ds(base, B)], dstb[q], semi[q])
        pltpu.async_copy(ea.at[pl.ds(base, B)], eab[q], semi[q])

    def wait_idx(j, q):
        base = wid * EPW + j * B
        pltpu.make_async_copy(src.at[pl.ds(base, B)], srcb[q], semi[q]).wait()
        pltpu.make_async_copy(dst.at[pl.ds(base, B)], dstb[q], semi[q]).wait()
        pltpu.make_async_copy(ea.at[pl.ds(base, B)], eab[q], semi[q]).wait()

    def issue_gather(r, q, p):
        pltpu.async_copy(tab.at[srcb[q]], rows[r], semg[r])
        pltpu.async_copy(adt.at[dstb[q]], adg[p], semg[r])

    def wait_gather(r, q, p):
        pltpu.make_async_copy(tab.at[srcb[q]], rows[r], semg[r]).wait()
        pltpu.make_async_copy(adt.at[dstb[q]], adg[p], semg[r]).wait()

    def issue_scatter(r, q, p):
        pltpu.async_copy(rows[r], accm.at[dstb[q]], sems[p], add=True)

    def wait_scatter(r, q, p):
        pltpu.make_async_copy(rows[r], accm.at[dstb[q]], sems[p]).wait()

    def compute(r, q, p):
        rw, dg, eg = rows[r], adg[p], eab[q]

        def alpha_grp(g, _):
            e0 = 16 * g
            eav_all = eg[pl.ds(e0, 16)]
            for i in range(8):
                rowi = e0 + arow_c[i]
                asv = plsc.load_gather(rw, [rowi, acols])
                adv = plsc.load_gather(dg, [rowi, lane8])
                eav = _vgather(eav_all, arow_c[i])
                tt = asv + adv + eav * c16v
                ex = jnp.exp(jnp.maximum(tt, 0.2 * tt))
                exb[pl.ds(8 * e0 + 16 * i, 16)] = ex
                plsc.store_scatter(rw, [rowi, acols], ex)
            return 0

        lax.fori_loop(0, B // 16, alpha_grp, 0)

        def msg_grp(g, _):
            for i in range(4):
                e = 8 * g + 2 * i
                exv = exb[pl.ds(8 * e, 16)]
                for k in range(HC // 16):
                    bv0 = _vgather(exv, bidx_e0[k])
                    bv1 = _vgather(exv, bidx_e1[k])
                    rw[e, pl.ds(16 * k, 16)] = rw[e, pl.ds(16 * k, 16)] * bv0
                    rw[e + 1, pl.ds(16 * k, 16)] = \
                        rw[e + 1, pl.ds(16 * k, 16)] * bv1
            return 0

        lax.fori_loop(0, B // 8, msg_grp, 0)

    _pipeline(issue_idx, wait_idx, issue_gather, wait_gather, compute,
              issue_scatter, wait_scatter)
    plsc.subcore_barrier()
    pltpu.sync_copy(accm.at[pl.ds(s * RPT, RPT)],
                    outm.at[c, pl.ds(s * RPT, RPT)])


_edges1 = functools.partial(
    pl.kernel,
    out_type=jax.ShapeDtypeStruct((NC, NPAD, ROW1), _f32),
    mesh=_MESH,
    compiler_params=_SC_PARAMS,
    scratch_types=[
        pltpu.VMEM_SHARED((NPAD, ROW1), _f32),
        pltpu.VMEM((B, ROW1), _f32),
        pltpu.VMEM((B, ROW1), _f32),
        pltpu.VMEM((B, ROW1), _f32),
        pltpu.VMEM((B, HEADS), _f32),
        pltpu.VMEM((B, HEADS), _f32),
        pltpu.VMEM((B * HEADS,), _f32),
        pltpu.VMEM((16,), _f32),
    ] + _IDX3 + _IDX3 + _IDX3 + _IDX3 + _SEM9,
)(_edges1_body)


def _edges2_body(tab2, ast2, adt2, src, dst, ea, c16, zm, outm,
                 accm, astab, adtab, rows0, rows1, rows2, exb, c16b,
                 sb0, db0, eb0, sb1, db1, eb1, sb2, db2, eb2, sb3, db3, eb3,
                 sg0, sg1, sg2, si0, si1, si2, si3, ss0, ss1):
    rows = [rows0, rows1, rows2]
    srcb, dstb = [sb0, sb1, sb2, sb3], [db0, db1, db2, db3]
    eab = [eb0, eb1, eb2, eb3]
    semg, semi, sems = [sg0, sg1, sg2], [si0, si1, si2, si3], [ss0, ss1]

    c = lax.axis_index("c")
    s = lax.axis_index("s")
    wid = c * NS + s
    pltpu.sync_copy(zm, accm.at[pl.ds(s * RPT, RPT)])
    pltpu.sync_copy(ast2, astab)
    pltpu.sync_copy(adt2, adtab)
    pltpu.sync_copy(c16, c16b)
    plsc.subcore_barrier()

    iota = lax.iota(jnp.int32, 16)
    zero16 = jnp.zeros((16,), jnp.int32)
    c16v = c16b[...]
    col40 = zero16 + NCLASS
    pick_c = [zero16 + i for i in range(16)]

    def issue_idx(j, q):
        base = wid * EPW + j * B
        pltpu.async_copy(src.at[pl.ds(base, B)], srcb[q], semi[q])
        pltpu.async_copy(dst.at[pl.ds(base, B)], dstb[q], semi[q])
        pltpu.async_copy(ea.at[pl.ds(base, B)], eab[q], semi[q])

    def wait_idx(j, q):
        base = wid * EPW + j * B
        pltpu.make_async_copy(src.at[pl.ds(base, B)], srcb[q], semi[q]).wait()
        pltpu.make_async_copy(dst.at[pl.ds(base, B)], dstb[q], semi[q]).wait()
        pltpu.make_async_copy(ea.at[pl.ds(base, B)], eab[q], semi[q]).wait()

    def issue_gather(r, q, p):
        pltpu.async_copy(tab2.at[srcb[q]], rows[r], semg[r])

    def wait_gather(r, q, p):
        pltpu.make_async_copy(tab2.at[srcb[q]], rows[r], semg[r]).wait()

    def issue_scatter(r, q, p):
        pltpu.async_copy(rows[r], accm.at[dstb[q]], sems[p], add=True)

    def wait_scatter(r, q, p):
        pltpu.make_async_copy(rows[r], accm.at[dstb[q]], sems[p]).wait()

    def compute(r, q, p):
        rw, sg, dg, eg = rows[r], srcb[q], dstb[q], eab[q]

        def alpha_grp(i, _):
            e0 = 16 * i
            srcv = sg[pl.ds(e0, 16)]
            dstv = dg[pl.ds(e0, 16)]
            eav = eg[pl.ds(e0, 16)]
            asv = plsc.load_gather(astab, [srcv])
            adv = plsc.load_gather(adtab, [dstv])
            tt = asv + adv + eav * c16v
            ex = jnp.exp(jnp.maximum(tt, 0.2 * tt))
            exb[pl.ds(e0, 16)] = ex
            return 0

        lax.fori_loop(0, B // 16, alpha_grp, 0)

        def msg_grp(g, _):
            e0 = 16 * g
            exv = exb[pl.ds(e0, 16)]
            for i in range(16):
                e = e0 + i
                bv = _vgather(exv, pick_c[i])
                for k in range(ROW2 // 16):
                    rw[e, pl.ds(16 * k, 16)] = rw[e, pl.ds(16 * k, 16)] * bv
            plsc.store_scatter(rw, [e0 + iota, col40], exv)
            return 0

        lax.fori_loop(0, B // 16, msg_grp, 0)

    _pipeline(issue_idx, wait_idx, issue_gather, wait_gather, compute,
              issue_scatter, wait_scatter)
    plsc.subcore_barrier()
    pltpu.sync_copy(accm.at[pl.ds(s * RPT, RPT)],
                    outm.at[c, pl.ds(s * RPT, RPT)])


_edges2 = functools.partial(
    pl.kernel,
    out_type=jax.ShapeDtypeStruct((NC, NPAD, ROW2), _f32),
    mesh=_MESH,
    compiler_params=_SC_PARAMS,
    scratch_types=[
        pltpu.VMEM_SHARED((NPAD, ROW2), _f32),
        pltpu.VMEM((N,), _f32),
        pltpu.VMEM((NPAD,), _f32),
        pltpu.VMEM((B, ROW2), _f32),
        pltpu.VMEM((B, ROW2), _f32),
        pltpu.VMEM((B, ROW2), _f32),
        pltpu.VMEM((B,), _f32),
        pltpu.VMEM((16,), _f32),
    ] + _IDX3 + _IDX3 + _IDX3 + _IDX3 + _SEM9,
)(_edges2_body)


def _pad_edges(a, fill):
    return jnp.concatenate(
        [a.reshape(NW, E // NW),
         jnp.full((NW, EPW - E // NW), fill, a.dtype)], axis=1).reshape(-1)


def kernel(x, edge_index, edge_attr, W1, att_src1, att_dst1, We1, att_e1, b1,
           W2, att_src2, att_dst2, We2, att_e2, b2):
    src = _pad_edges(edge_index[0], 0)
    dst = _pad_edges(edge_index[1], N)
    ea = _pad_edges(edge_attr[:, 0], 0.0)

    eye8 = jnp.repeat(jnp.eye(HEADS, dtype=_f32), NHID, axis=0)
    Asrc = eye8 * att_src1.reshape(HC, 1)
    Adst = eye8 * att_dst1.reshape(HC, 1)
    exp8 = jnp.repeat(jnp.eye(HEADS, dtype=_f32), NHID, axis=1)
    c1 = (We1.reshape(HEADS, NHID) * att_e1[0]).sum(-1)
    c1_16 = jnp.tile(c1, 2)
    c2 = (We2.reshape(NCLASS) * att_e2[0, 0]).sum()
    c2_16 = jnp.full((16,), c2, _f32)
    zm1 = jnp.zeros((RPT, ROW1), _f32)
    zm2 = jnp.zeros((RPT, ROW2), _f32)

    eam = _ea_mean(edge_attr[:, 0])
    tab, ad = _prep1(x, W1, Asrc, Adst)
    ad_p = jnp.pad(ad, ((0, NPAD - N), (0, 0)))
    acc = _edges1(tab, ad_p, src, dst, ea, c1_16, zm1)
    tab2, as2, ad2 = _mid(tab, acc, ad, eam,
                          c1.reshape(1, HEADS), b1.reshape(1, HC), exp8, W2,
                          att_src2.reshape(NCLASS, 1),
                          att_dst2.reshape(NCLASS, 1))
    ad2_p = jnp.pad(ad2.reshape(N), (0, NPAD - N))
    acc2 = _edges2(tab2, as2.reshape(N), ad2_p,
                   src, dst, ea, c2_16, zm2)
    return _final(tab2, as2, acc2, ad2, eam, c2.reshape(1, 1),
                  b2.reshape(1, NCLASS))


# speedup vs baseline: 1.0024x; 1.0024x over previous
import functools

import jax
import jax.numpy as jnp
from jax import lax
from jax.experimental import pallas as pl
from jax.experimental.pallas import tpu as pltpu
from jax.experimental.pallas import tpu_sc as plsc

N, E, F_IN = 10000, 320000, 128
HEADS, NHID, NCLASS = 8, 8, 40
HC = HEADS * NHID
ROW1 = 72
ROW2 = 48
NC, NS = 2, 16
NW = NC * NS
B = 272
NCHUNK = 38
EPW = B * NCHUNK
NPAD = 10240
RPT = NPAD // NS

_f32 = jnp.float32


def _ea_sum_body(ea_ref, out_ref):
    out_ref[...] = jnp.sum(ea_ref[...]).reshape(1, 1) * (1.0 / E)


def _ea_mean(ea):
    return pl.pallas_call(
        _ea_sum_body,
        out_shape=jax.ShapeDtypeStruct((1, 1), _f32),
        in_specs=[pl.BlockSpec((2500, 128), lambda: (0, 0))],
        out_specs=pl.BlockSpec((1, 1), lambda: (0, 0)),
    )(ea.reshape(2500, 128))


def _prep1_body(x_ref, w_ref, asrc_ref, adst_ref, tab_ref, ad_ref):
    xs = jnp.dot(x_ref[...], w_ref[...], preferred_element_type=_f32)
    tab_ref[:, :HC] = xs
    tab_ref[:, HC:] = jnp.dot(xs, asrc_ref[...], preferred_element_type=_f32)
    ad_ref[...] = jnp.dot(xs, adst_ref[...], preferred_element_type=_f32)


def _prep1(x, W1, Asrc, Adst):
    R = 1000
    return pl.pallas_call(
        _prep1_body,
        grid=(N // R,),
        out_shape=[jax.ShapeDtypeStruct((N, ROW1), _f32),
                   jax.ShapeDtypeStruct((N, HEADS), _f32)],
        in_specs=[pl.BlockSpec((R, F_IN), lambda i: (i, 0)),
                  pl.BlockSpec((F_IN, HC), lambda i: (0, 0)),
                  pl.BlockSpec((HC, HEADS), lambda i: (0, 0)),
                  pl.BlockSpec((HC, HEADS), lambda i: (0, 0))],
        out_specs=[pl.BlockSpec((R, ROW1), lambda i: (i, 0)),
                   pl.BlockSpec((R, HEADS), lambda i: (i, 0))],
    )(x, W1, Asrc, Adst)


def _mid_body(tab_ref, accA_ref, accB_ref,
              ad_ref, eam_ref, c1_ref, b1_ref, exp8_ref, w2_ref, a2s_ref,
              a2d_ref, tab2_ref, as2_ref, ad2_ref):
    xs = tab_ref[:, :HC]
    al = tab_ref[:, HC:] + ad_ref[...] + eam_ref[0, 0] * c1_ref[...]
    ex = jnp.exp(jnp.maximum(al, 0.2 * al))
    exp8 = exp8_ref[...]
    num = accA_ref[0, :, :HC] + accB_ref[0, :, :HC] \
        + jnp.dot(ex, exp8, preferred_element_type=_f32) * xs
    den = accA_ref[0, :, HC:] + accB_ref[0, :, HC:] + ex
    h = num / jnp.dot(den, exp8, preferred_element_type=_f32) + b1_ref[...]
    h = jnp.where(h > 0, h, jnp.exp(jnp.minimum(h, 0.0)) - 1.0)
    xs2 = jnp.dot(h, w2_ref[...], preferred_element_type=_f32)
    tab2_ref[:, :NCLASS] = xs2
    tab2_ref[:, NCLASS:] = jnp.zeros_like(tab2_ref[:, NCLASS:])
    as2_ref[...] = jnp.dot(xs2, a2s_ref[...], preferred_element_type=_f32)
    ad2_ref[...] = jnp.dot(xs2, a2d_ref[...], preferred_element_type=_f32)


def _mid(tab, acc, ad, eam, c1, b1, exp8, W2, a2s, a2d):
    R = 1000
    return pl.pallas_call(
        _mid_body,
        grid=(N // R,),
        out_shape=[jax.ShapeDtypeStruct((N, ROW2), _f32),
                   jax.ShapeDtypeStruct((N, 1), _f32),
                   jax.ShapeDtypeStruct((N, 1), _f32)],
        in_specs=[pl.BlockSpec((R, ROW1), lambda i: (i, 0)),
                  pl.BlockSpec((1, R, ROW1), lambda i: (0, i, 0)),
                  pl.BlockSpec((1, R, ROW1), lambda i: (1, i, 0)),
                  pl.BlockSpec((R, HEADS), lambda i: (i, 0)),
                  pl.BlockSpec((1, 1), lambda i: (0, 0)),
                  pl.BlockSpec((1, HEADS), lambda i: (0, 0)),
                  pl.BlockSpec((1, HC), lambda i: (0, 0)),
                  pl.BlockSpec((HEADS, HC), lambda i: (0, 0)),
                  pl.BlockSpec((HC, NCLASS), lambda i: (0, 0)),
                  pl.BlockSpec((NCLASS, 1), lambda i: (0, 0)),
                  pl.BlockSpec((NCLASS, 1), lambda i: (0, 0))],
        out_specs=[pl.BlockSpec((R, ROW2), lambda i: (i, 0)),
                   pl.BlockSpec((R, 1), lambda i: (i, 0)),
                   pl.BlockSpec((R, 1), lambda i: (i, 0))],
    )(tab, acc, acc, ad, eam, c1, b1, exp8, W2, a2s, a2d)


def _final_body(tab2_ref, as2_ref, accA_ref, accB_ref,
                ad2_ref, eam_ref, c2_ref, b2_ref, out_ref):
    xs2 = tab2_ref[:, :NCLASS]
    al = as2_ref[...] + ad2_ref[...] + eam_ref[0, 0] * c2_ref[0, 0]
    ex = jnp.exp(jnp.maximum(al, 0.2 * al))
    num = accA_ref[0, :, :NCLASS] + accB_ref[0, :, :NCLASS] + ex * xs2
    den = accA_ref[0, :, NCLASS:NCLASS + 1] \
        + accB_ref[0, :, NCLASS:NCLASS + 1] + ex
    o = num / den + b2_ref[...]
    m = jnp.max(o, axis=1, keepdims=True)
    s = jnp.sum(jnp.exp(o - m), axis=1, keepdims=True)
    out_ref[...] = o - m - jnp.log(s)


def _final(tab2, as2, acc, ad2, eam, c2, b2):
    R = 1000
    return pl.pallas_call(
        _final_body,
        grid=(N // R,),
        out_shape=jax.ShapeDtypeStruct((N, NCLASS), _f32),
        in_specs=[pl.BlockSpec((R, ROW2), lambda i: (i, 0)),
                  pl.BlockSpec((R, 1), lambda i: (i, 0)),
                  pl.BlockSpec((1, R, ROW2), lambda i: (0, i, 0)),
                  pl.BlockSpec((1, R, ROW2), lambda i: (1, i, 0)),
                  pl.BlockSpec((R, 1), lambda i: (i, 0)),
                  pl.BlockSpec((1, 1), lambda i: (0, 0)),
                  pl.BlockSpec((1, 1), lambda i: (0, 0)),
                  pl.BlockSpec((1, NCLASS), lambda i: (0, 0))],
        out_specs=pl.BlockSpec((R, NCLASS), lambda i: (i, 0)),
    )(tab2, as2, acc, acc, ad2, eam, c2, b2)


def _pipeline(issue_idx, wait_idx, issue_gather, wait_gather, compute,
              issue_scatter, wait_scatter):
    issue_idx(0, 0)
    issue_idx(1, 1)
    wait_idx(0, 0)
    issue_gather(0, 0, 0)
    for j in (0, 1):
        wait_idx(j + 1, j + 1)
        issue_gather((j + 1) % 3, j + 1, (j + 1) % 2)
        issue_idx(j + 2, j + 2)
        wait_gather(j % 3, j % 4, j % 2)
        compute(j % 3, j % 4, j % 2)
        issue_scatter(j % 3, j % 4, j % 2)

    def superstep(ss, _):
        for i in range(12):
            j = 12 * ss + 2 + i
            r, q, p = (2 + i) % 3, (2 + i) % 4, i % 2
            wait_scatter(i % 3, i % 4, p)

            def _prefetch(r1=(3 + i) % 3, q1=(3 + i) % 4, p1=(i + 1) % 2,
                          jj=j + 1):
                wait_idx(jj, q1)
                issue_gather(r1, q1, p1)

            def _nextidx(jj=j + 2, q2=i % 4):
                issue_idx(jj, q2)

            pl.when(j + 1 < NCHUNK)(_prefetch)
            pl.when(j + 2 < NCHUNK)(_nextidx)
            wait_gather(r, q, p)
            compute(r, q, p)
            issue_scatter(r, q, p)
        return 0

    lax.fori_loop(0, (NCHUNK - 2) // 12, superstep, 0)
    wait_scatter(0, 0, 0)
    wait_scatter(1, 1, 1)


def _vgather(v, idx):
    return lax.gather(
        v, idx.reshape(16, 1),
        lax.GatherDimensionNumbers(offset_dims=(), collapsed_slice_dims=(0,),
                                   start_index_map=(0,)),
        slice_sizes=(1,), mode=lax.GatherScatterMode.PROMISE_IN_BOUNDS)


_MESH = plsc.VectorSubcoreMesh(core_axis_name="c", subcore_axis_name="s",
                               num_cores=NC, num_subcores=NS)
_SC_PARAMS = pltpu.CompilerParams(needs_layout_passes=False,
                                  use_tc_tiling_on_sc=False)
_IDX3 = [pltpu.VMEM((B,), jnp.int32), pltpu.VMEM((B,), jnp.int32),
         pltpu.VMEM((B,), _f32)]
_SEM9 = [pltpu.SemaphoreType.DMA] * 9


def _edges1_body(tab, adt, src, dst, ea, c16, zm, outm,
                 accm, rows0, rows1, rows2, adg0, adg1, exb, c16b,
                 sb0, db0, eb0, sb1, db1, eb1, sb2, db2, eb2, sb3, db3, eb3,
                 sg0, sg1, sg2, si0, si1, si2, si3, ss0, ss1):
    rows = [rows0, rows1, rows2]
    adg = [adg0, adg1]
    srcb, dstb = [sb0, sb1, sb2, sb3], [db0, db1, db2, db3]
    eab = [eb0, eb1, eb2, eb3]
    semg, semi, sems = [sg0, sg1, sg2], [si0, si1, si2, si3], [ss0, ss1]

    c = lax.axis_index("c")
    s = lax.axis_index("s")
    wid = c * NS + s
    pltpu.sync_copy(zm, accm.at[pl.ds(s * RPT, RPT)])
    pltpu.sync_copy(c16, c16b)
    plsc.subcore_barrier()

    iota = lax.iota(jnp.int32, 16)
    lane8 = jnp.bitwise_and(iota, 7)
    pair8 = jnp.right_shift(iota, 3)
    c16v = c16b[...]
    acols = HC + lane8
    arow_c = [2 * i + pair8 for i in range(8)]
    bidx_e0 = [2 * k + pair8 for k in range(4)]
    bidx_e1 = [8 + 2 * k + pair8 for k in range(4)]

    def issue_idx(j, q):
        base = wid * EPW + j * B
        pltpu.async_copy(src.at[pl.ds(base, B)], srcb[q], semi[q])
        pltpu.async_copy(dst.at[pl.ds(base, B)], dstb[q], semi[q])
        pltpu.async_copy(ea.at[pl.ds(base, B)], eab[q], semi[q])

    def wait_idx(j, q):
        base = wid * EPW + j * B
        pltpu.make_async_copy(src.at[pl.ds(base, B)], srcb[q], semi[q]).wait()
        pltpu.make_async_copy(dst.at[pl.ds(base, B)], dstb[q], semi[q]).wait()
        pltpu.make_async_copy(ea.at[pl.ds(base, B)], eab[q], semi[q]).wait()

    def issue_gather(r, q, p):
        pltpu.async_copy(tab.at[srcb[q]], rows[r], semg[r])
        pltpu.async_copy(adt.at[dstb[q]], adg[p], semg[r])

    def wait_gather(r, q, p):
        pltpu.make_async_copy(tab.at[srcb[q]], rows[r], semg[r]).wait()
        pltpu.make_async_copy(adt.at[dstb[q]], adg[p], semg[r]).wait()

    def issue_scatter(r, q, p):
        pltpu.async_copy(rows[r], accm.at[dstb[q]], sems[p], add=True)

    def wait_scatter(r, q, p):
        pltpu.make_async_copy(rows[r], accm.at[dstb[q]], sems[p]).wait()

    def compute(r, q, p):
        rw, dg, eg = rows[r], adg[p], eab[q]

        def alpha_grp(g, _):
            e0 = 16 * g
            eav_all = eg[pl.ds(e0, 16)]
            for i in range(8):
                rowi = e0 + arow_c[i]
                asv = plsc.load_gather(rw, [rowi, acols])
                adv = plsc.load_gather(dg, [rowi, lane8])
                eav = _vgather(eav_all, arow_c[i])
                tt = asv + adv + eav * c16v
                ex = jnp.exp(jnp.maximum(tt, 0.2 * tt))
                exb[pl.ds(8 * e0 + 16 * i, 16)] = ex
                plsc.store_scatter(rw, [rowi, acols], ex)
            return 0

        lax.fori_loop(0, B // 16, alpha_grp, 0)

        def msg_grp(g, _):
            for i in range(4):
                e = 8 * g + 2 * i
                exv = exb[pl.ds(8 * e, 16)]
                for k in range(HC // 16):
                    bv0 = _vgather(exv, bidx_e0[k])
                    bv1 = _vgather(exv, bidx_e1[k])
                    rw[e, pl.ds(16 * k, 16)] = rw[e, pl.ds(16 * k, 16)] * bv0
                    rw[e + 1, pl.ds(16 * k, 16)] = \
                        rw[e + 1, pl.ds(16 * k, 16)] * bv1
            return 0

        lax.fori_loop(0, B // 8, msg_grp, 0)

    _pipeline(issue_idx, wait_idx, issue_gather, wait_gather, compute,
              issue_scatter, wait_scatter)
    plsc.subcore_barrier()
    pltpu.sync_copy(accm.at[pl.ds(s * RPT, RPT)],
                    outm.at[c, pl.ds(s * RPT, RPT)])


_edges1 = functools.partial(
    pl.kernel,
    out_type=jax.ShapeDtypeStruct((NC, NPAD, ROW1), _f32),
    mesh=_MESH,
    compiler_params=_SC_PARAMS,
    scratch_types=[
        pltpu.VMEM_SHARED((NPAD, ROW1), _f32),
        pltpu.VMEM((B, ROW1), _f32),
        pltpu.VMEM((B, ROW1), _f32),
        pltpu.VMEM((B, ROW1), _f32),
        pltpu.VMEM((B, HEADS), _f32),
        pltpu.VMEM((B, HEADS), _f32),
        pltpu.VMEM((B * HEADS,), _f32),
        pltpu.VMEM((16,), _f32),
    ] + _IDX3 + _IDX3 + _IDX3 + _IDX3 + _SEM9,
)(_edges1_body)


def _edges2_body(tab2, ast2, adt2, src, dst, ea, c16, zm, outm,
                 accm, astab, adtab, rows0, rows1, rows2, exb, c16b,
                 sb0, db0, eb0, sb1, db1, eb1, sb2, db2, eb2, sb3, db3, eb3,
                 sg0, sg1, sg2, si0, si1, si2, si3, ss0, ss1):
    rows = [rows0, rows1, rows2]
    srcb, dstb = [sb0, sb1, sb2, sb3], [db0, db1, db2, db3]
    eab = [eb0, eb1, eb2, eb3]
    semg, semi, sems = [sg0, sg1, sg2], [si0, si1, si2, si3], [ss0, ss1]

    c = lax.axis_index("c")
    s = lax.axis_index("s")
    wid = c * NS + s
    pltpu.sync_copy(zm, accm.at[pl.ds(s * RPT, RPT)])
    pltpu.sync_copy(ast2, astab)
    pltpu.sync_copy(adt2, adtab)
    pltpu.sync_copy(c16, c16b)
    plsc.subcore_barrier()

    iota = lax.iota(jnp.int32, 16)
    zero16 = jnp.zeros((16,), jnp.int32)
    c16v = c16b[...]
    col40 = zero16 + NCLASS
    pick_c = [zero16 + i for i in range(16)]

    def issue_idx(j, q):
        base = wid * EPW + j * B
        pltpu.async_copy(src.at[pl.ds(base, B)], srcb[q], semi[q])
        pltpu.async_copy(dst.at[pl.ds(base, B)], dstb[q], semi[q])
        pltpu.async_copy(ea.at[pl.ds(base, B)], eab[q], semi[q])

    def wait_idx(j, q):
        base = wid * EPW + j * B
        pltpu.make_async_copy(src.at[pl.ds(base, B)], srcb[q], semi[q]).wait()
        pltpu.make_async_copy(dst.at[pl.ds(base, B)], dstb[q], semi[q]).wait()
        pltpu.make_async_copy(ea.at[pl.ds(base, B)], eab[q], semi[q]).wait()

    def issue_gather(r, q, p):
        pltpu.async_copy(tab2.at[srcb[q]], rows[r], semg[r])

    def wait_gather(r, q, p):
        pltpu.make_async_copy(tab2.at[srcb[q]], rows[r], semg[r]).wait()

    def issue_scatter(r, q, p):
        pltpu.async_copy(rows[r], accm.at[dstb[q]], sems[p], add=True)

    def wait_scatter(r, q, p):
        pltpu.make_async_copy(rows[r], accm.at[dstb[q]], sems[p]).wait()

    def compute(r, q, p):
        rw, sg, dg, eg = rows[r], srcb[q], dstb[q], eab[q]

        def alpha_grp(i, _):
            e0 = 16 * i
            srcv = sg[pl.ds(e0, 16)]
            dstv = dg[pl.ds(e0, 16)]
            eav = eg[pl.ds(e0, 16)]
            asv = plsc.load_gather(astab, [srcv])
            adv = plsc.load_gather(adtab, [dstv])
            tt = asv + adv + eav * c16v
            ex = jnp.exp(jnp.maximum(tt, 0.2 * tt))
            exb[pl.ds(e0, 16)] = ex
            return 0

        lax.fori_loop(0, B // 16, alpha_grp, 0)

        def msg_grp(g, _):
            e0 = 16 * g
            exv = exb[pl.ds(e0, 16)]
            for i in range(16):
                e = e0 + i
                bv = _vgather(exv, pick_c[i])
                for k in range(ROW2 // 16):
                    rw[e, pl.ds(16 * k, 16)] = rw[e, pl.ds(16 * k, 16)] * bv
            plsc.store_scatter(rw, [e0 + iota, col40], exv)
            return 0

        lax.fori_loop(0, B // 16, msg_grp, 0)

    _pipeline(issue_idx, wait_idx, issue_gather, wait_gather, compute,
              issue_scatter, wait_scatter)
    plsc.subcore_barrier()
    pltpu.sync_copy(accm.at[pl.ds(s * RPT, RPT)],
                    outm.at[c, pl.ds(s * RPT, RPT)])


_edges2 = functools.partial(
    pl.kernel,
    out_type=jax.ShapeDtypeStruct((NC, NPAD, ROW2), _f32),
    mesh=_MESH,
    compiler_params=_SC_PARAMS,
    scratch_types=[
        pltpu.VMEM_SHARED((NPAD, ROW2), _f32),
        pltpu.VMEM((N,), _f32),
        pltpu.VMEM((NPAD,), _f32),
        pltpu.VMEM((B, ROW2), _f32),
        pltpu.VMEM((B, ROW2), _f32),
        pltpu.VMEM((B, ROW2), _f32),
        pltpu.VMEM((B,), _f32),
        pltpu.VMEM((16,), _f32),
    ] + _IDX3 + _IDX3 + _IDX3 + _IDX3 + _SEM9,
)(_edges2_body)


def _pad_edges(a, fill):
    pad = jnp.broadcast_to(fill, (NW, EPW - E // NW)).astype(a.dtype)
    return jnp.concatenate([a.reshape(NW, E // NW), pad], axis=1).reshape(-1)


def kernel(x, edge_index, edge_attr, W1, att_src1, att_dst1, We1, att_e1, b1,
           W2, att_src2, att_dst2, We2, att_e2, b2):
    npadd = EPW - E // NW
    src = _pad_edges(edge_index[0], jnp.zeros((npadd,), jnp.int32))
    dst = _pad_edges(edge_index[1],
                     N + jnp.arange(npadd, dtype=jnp.int32) % (NPAD - N))
    ea = _pad_edges(edge_attr[:, 0], jnp.zeros((npadd,), _f32))

    eye8 = jnp.repeat(jnp.eye(HEADS, dtype=_f32), NHID, axis=0)
    Asrc = eye8 * att_src1.reshape(HC, 1)
    Adst = eye8 * att_dst1.reshape(HC, 1)
    exp8 = jnp.repeat(jnp.eye(HEADS, dtype=_f32), NHID, axis=1)
    c1 = (We1.reshape(HEADS, NHID) * att_e1[0]).sum(-1)
    c1_16 = jnp.tile(c1, 2)
    c2 = (We2.reshape(NCLASS) * att_e2[0, 0]).sum()
    c2_16 = jnp.full((16,), c2, _f32)
    zm1 = jnp.zeros((RPT, ROW1), _f32)
    zm2 = jnp.zeros((RPT, ROW2), _f32)

    eam = _ea_mean(edge_attr[:, 0])
    tab, ad = _prep1(x, W1, Asrc, Adst)
    ad_p = jnp.pad(ad, ((0, NPAD - N), (0, 0)))
    acc = _edges1(tab, ad_p, src, dst, ea, c1_16, zm1)
    tab2, as2, ad2 = _mid(tab, acc, ad, eam,
                          c1.reshape(1, HEADS), b1.reshape(1, HC), exp8, W2,
                          att_src2.reshape(NCLASS, 1),
                          att_dst2.reshape(NCLASS, 1))
    ad2_p = jnp.pad(ad2.reshape(N), (0, NPAD - N))
    acc2 = _edges2(tab2, as2.reshape(N), ad2_p,
                   src, dst, ea, c2_16, zm2)
    return _final(tab2, as2, acc2, ad2, eam, c2.reshape(1, 1),
                  b2.reshape(1, NCLASS))


# speedup vs baseline: 1.7511x; 1.7470x over previous
import functools

import jax
import jax.numpy as jnp
from jax import lax
from jax.experimental import pallas as pl
from jax.experimental.pallas import tpu as pltpu
from jax.experimental.pallas import tpu_sc as plsc

N, E, F_IN = 10000, 320000, 128
HEADS, NHID, NCLASS = 8, 8, 40
HC = HEADS * NHID
ROW1 = HC + HEADS
ROW2 = 48
NC, NS = 2, 16
NW = NC * NS
EPW = E // NW
B = 400
NCHUNK = EPW // B
NPAD = 10240
RPT = NPAD // NS

_f32 = jnp.float32


def _ea_sum_body(ea_ref, out_ref):
    out_ref[...] = jnp.sum(ea_ref[...]).reshape(1, 1) * (1.0 / E)


def _ea_mean(ea):
    return pl.pallas_call(
        _ea_sum_body,
        out_shape=jax.ShapeDtypeStruct((1, 1), _f32),
        in_specs=[pl.BlockSpec((2500, 128), lambda: (0, 0))],
        out_specs=pl.BlockSpec((1, 1), lambda: (0, 0)),
    )(ea.reshape(2500, 128))


def _prep1_body(x_ref, w_ref, asrc_ref, adst_ref, tab_ref, ad_ref):
    xs = jnp.dot(x_ref[...], w_ref[...], preferred_element_type=_f32)
    tab_ref[:, :HC] = xs
    tab_ref[:, HC:] = jnp.dot(xs, asrc_ref[...], preferred_element_type=_f32)
    ad_ref[...] = jnp.dot(xs, adst_ref[...], preferred_element_type=_f32)


def _prep1(x, W1, Asrc, Adst):
    R = 1000
    return pl.pallas_call(
        _prep1_body,
        grid=(N // R,),
        out_shape=[jax.ShapeDtypeStruct((N, ROW1), _f32),
                   jax.ShapeDtypeStruct((N, HEADS), _f32)],
        in_specs=[pl.BlockSpec((R, F_IN), lambda i: (i, 0)),
                  pl.BlockSpec((F_IN, HC), lambda i: (0, 0)),
                  pl.BlockSpec((HC, HEADS), lambda i: (0, 0)),
                  pl.BlockSpec((HC, HEADS), lambda i: (0, 0))],
        out_specs=[pl.BlockSpec((R, ROW1), lambda i: (i, 0)),
                   pl.BlockSpec((R, HEADS), lambda i: (i, 0))],
    )(x, W1, Asrc, Adst)


def _mid_body(tab_ref, accA_ref, accB_ref, ad_ref, eam_ref, c1_ref, b1_ref,
              exp8_ref, w2_ref, a2s_ref, a2d_ref, tab2_ref, ad2_ref):
    xs = tab_ref[:, :HC]
    al = tab_ref[:, HC:] + ad_ref[...] + eam_ref[0, 0] * c1_ref[...]
    ex = jnp.exp(jnp.maximum(al, 0.2 * al))
    exp8 = exp8_ref[...]
    num = accA_ref[0, :, :HC] + accB_ref[0, :, :HC] \
        + jnp.dot(ex, exp8, preferred_element_type=_f32) * xs
    den = accA_ref[0, :, HC:] + accB_ref[0, :, HC:] + ex
    h = num / jnp.dot(den, exp8, preferred_element_type=_f32) + b1_ref[...]
    h = jnp.where(h > 0, h, jnp.exp(jnp.minimum(h, 0.0)) - 1.0)
    xs2 = jnp.dot(h, w2_ref[...], preferred_element_type=_f32)
    tab2_ref[:, :NCLASS] = xs2
    tab2_ref[:, NCLASS:NCLASS + 1] = jnp.dot(xs2, a2s_ref[...],
                                             preferred_element_type=_f32)
    tab2_ref[:, NCLASS + 1:] = jnp.zeros_like(tab2_ref[:, NCLASS + 1:])
    ad2_ref[...] = jnp.dot(xs2, a2d_ref[...], preferred_element_type=_f32)


def _mid(tab, acc, ad, eam, c1, b1, exp8, W2, a2s, a2d):
    R = 1000
    return pl.pallas_call(
        _mid_body,
        grid=(N // R,),
        out_shape=[jax.ShapeDtypeStruct((N, ROW2), _f32),
                   jax.ShapeDtypeStruct((N, 1), _f32)],
        in_specs=[pl.BlockSpec((R, ROW1), lambda i: (i, 0)),
                  pl.BlockSpec((1, R, ROW1), lambda i: (0, i, 0)),
                  pl.BlockSpec((1, R, ROW1), lambda i: (1, i, 0)),
                  pl.BlockSpec((R, HEADS), lambda i: (i, 0)),
                  pl.BlockSpec((1, 1), lambda i: (0, 0)),
                  pl.BlockSpec((1, HEADS), lambda i: (0, 0)),
                  pl.BlockSpec((1, HC), lambda i: (0, 0)),
                  pl.BlockSpec((HEADS, HC), lambda i: (0, 0)),
                  pl.BlockSpec((HC, NCLASS), lambda i: (0, 0)),
                  pl.BlockSpec((NCLASS, 1), lambda i: (0, 0)),
                  pl.BlockSpec((NCLASS, 1), lambda i: (0, 0))],
        out_specs=[pl.BlockSpec((R, ROW2), lambda i: (i, 0)),
                   pl.BlockSpec((R, 1), lambda i: (i, 0))],
    )(tab, acc, acc, ad, eam, c1, b1, exp8, W2, a2s, a2d)


def _final_body(tab2_ref, accA_ref, accB_ref, ad2_ref, eam_ref, c2_ref,
                b2_ref, out_ref):
    xs2 = tab2_ref[:, :NCLASS]
    al = tab2_ref[:, NCLASS:NCLASS + 1] + ad2_ref[...] + eam_ref[0, 0] * c2_ref[0, 0]
    ex = jnp.exp(jnp.maximum(al, 0.2 * al))
    num = accA_ref[0, :, :NCLASS] + accB_ref[0, :, :NCLASS] + ex * xs2
    den = accA_ref[0, :, NCLASS:NCLASS + 1] + accB_ref[0, :, NCLASS:NCLASS + 1] + ex
    o = num / den + b2_ref[...]
    m = jnp.max(o, axis=1, keepdims=True)
    s = jnp.sum(jnp.exp(o - m), axis=1, keepdims=True)
    out_ref[...] = o - m - jnp.log(s)


def _final(tab2, acc2, ad2, eam, c2, b2):
    R = 1000
    return pl.pallas_call(
        _final_body,
        grid=(N // R,),
        out_shape=jax.ShapeDtypeStruct((N, NCLASS), _f32),
        in_specs=[pl.BlockSpec((R, ROW2), lambda i: (i, 0)),
                  pl.BlockSpec((1, R, ROW2), lambda i: (0, i, 0)),
                  pl.BlockSpec((1, R, ROW2), lambda i: (1, i, 0)),
                  pl.BlockSpec((R, 1), lambda i: (i, 0)),
                  pl.BlockSpec((1, 1), lambda i: (0, 0)),
                  pl.BlockSpec((1, 1), lambda i: (0, 0)),
                  pl.BlockSpec((1, NCLASS), lambda i: (0, 0))],
        out_specs=pl.BlockSpec((R, NCLASS), lambda i: (i, 0)),
    )(tab2, acc2, acc2, ad2, eam, c2, b2)


def _pipeline(issue_idx, wait_idx, issue_gather, wait_gather, compute,
              issue_scatter, wait_scatter):
    issue_idx(0, 0)
    issue_idx(1, 1)
    wait_idx(0, 0)
    issue_gather(0, 0)

    def superstep(ss, _):
        for i in range(6):
            j = 6 * ss + i
            b, t = i % 2, i % 3
            pl.when(j >= 1)(lambda b1=(i + 1) % 2, t2=(i + 2) % 3:
                            wait_scatter(b1, t2))
            wait_idx(j + 1, (i + 1) % 3)
            issue_gather((i + 1) % 2, (i + 1) % 3)
            pl.when(j + 2 < NCHUNK)(lambda jj=j + 2, t2=(i + 2) % 3:
                                    issue_idx(jj, t2))
            wait_gather(b, t)
            compute(b, t)
            issue_scatter(b, t)
        return 0

    lax.fori_loop(0, (NCHUNK - 1) // 6, superstep, 0)
    wait_scatter(1, 2)
    wait_gather(0, 0)
    compute(0, 0)
    issue_scatter(0, 0)
    wait_scatter(0, 0)


def _vgather(v, idx):
    return lax.gather(
        v, idx.reshape(16, 1),
        lax.GatherDimensionNumbers(offset_dims=(), collapsed_slice_dims=(0,),
                                   start_index_map=(0,)),
        slice_sizes=(1,), mode=lax.GatherScatterMode.PROMISE_IN_BOUNDS)


_MESH = plsc.VectorSubcoreMesh(core_axis_name="c", subcore_axis_name="s",
                               num_cores=NC, num_subcores=NS)
_SC_PARAMS = pltpu.CompilerParams(needs_layout_passes=False,
                                  use_tc_tiling_on_sc=False)


def _edges1_body(tab, ad, src, dst, ea, c16, zeros, out, accum,
                 rows0, rows1, adg0, adg1, exb, c16b,
                 sb0, sb1, sb2, db0, db1, db2, eb0, eb1, eb2,
                 sg0, sg1, si0, si1, si2, ss0, ss1):
    rows = [rows0, rows1]
    adg = [adg0, adg1]
    srcb, dstb, eab = [sb0, sb1, sb2], [db0, db1, db2], [eb0, eb1, eb2]
    semg, semi, sems = [sg0, sg1], [si0, si1, si2], [ss0, ss1]

    c = lax.axis_index("c")
    s = lax.axis_index("s")
    wid = c * NS + s
    pltpu.sync_copy(zeros, accum.at[pl.ds(s * RPT, RPT)])
    pltpu.sync_copy(c16, c16b)
    plsc.subcore_barrier()

    iota = lax.iota(jnp.int32, 16)
    lane8 = jnp.bitwise_and(iota, 7)
    pair8 = jnp.right_shift(iota, 3)
    c16v = c16b[...]
    acols = HC + lane8
    arow_c = [2 * i + pair8 for i in range(8)]
    eidx_c = [2 * i + pair8 for i in range(8)]
    bidx_e0 = [2 * k + pair8 for k in range(4)]
    bidx_e1 = [8 + 2 * k + pair8 for k in range(4)]

    def issue_idx(j, t):
        base = wid * EPW + j * B
        pltpu.async_copy(src.at[pl.ds(base, B)], srcb[t], semi[t])
        pltpu.async_copy(dst.at[pl.ds(base, B)], dstb[t], semi[t])
        pltpu.async_copy(ea.at[pl.ds(base, B)], eab[t], semi[t])

    def wait_idx(j, t):
        base = wid * EPW + j * B
        pltpu.make_async_copy(src.at[pl.ds(base, B)], srcb[t], semi[t]).wait()
        pltpu.make_async_copy(dst.at[pl.ds(base, B)], dstb[t], semi[t]).wait()
        pltpu.make_async_copy(ea.at[pl.ds(base, B)], eab[t], semi[t]).wait()

    def issue_gather(b, t):
        pltpu.async_copy(tab.at[srcb[t]], rows[b], semg[b])
        pltpu.async_copy(ad.at[dstb[t]], adg[b], semg[b])

    def wait_gather(b, t):
        pltpu.make_async_copy(tab.at[srcb[t]], rows[b], semg[b]).wait()
        pltpu.make_async_copy(ad.at[dstb[t]], adg[b], semg[b]).wait()

    def issue_scatter(b, t):
        pltpu.async_copy(rows[b], accum.at[dstb[t]], sems[b], add=True)

    def wait_scatter(b, t):
        pltpu.make_async_copy(rows[b], accum.at[dstb[t]], sems[b]).wait()

    def compute(b, t):
        rw, ag, eg = rows[b], adg[b], eab[t]

        def alpha_grp(q, _):
            e0 = 16 * q
            eav_all = eg[pl.ds(e0, 16)]
            for i in range(8):
                rowi = e0 + arow_c[i]
                asg = plsc.load_gather(rw, [rowi, acols])
                adv = plsc.load_gather(ag, [rowi, lane8])
                eav = _vgather(eav_all, eidx_c[i])
                tt = asg + adv + eav * c16v
                ex = jnp.exp(jnp.maximum(tt, 0.2 * tt))
                exb[pl.ds(8 * e0 + 16 * i, 16)] = ex
                plsc.store_scatter(rw, [rowi, acols], ex)
            return 0

        lax.fori_loop(0, B // 16, alpha_grp, 0)

        def msg_grp(g, _):
            for i in range(4):
                e = 8 * g + 2 * i
                exv = exb[pl.ds(8 * e, 16)]
                for k in range(HC // 16):
                    bv0 = _vgather(exv, bidx_e0[k])
                    bv1 = _vgather(exv, bidx_e1[k])
                    rw[e, pl.ds(16 * k, 16)] = rw[e, pl.ds(16 * k, 16)] * bv0
                    rw[e + 1, pl.ds(16 * k, 16)] = rw[e + 1, pl.ds(16 * k, 16)] * bv1
            return 0

        lax.fori_loop(0, B // 8, msg_grp, 0)

    _pipeline(issue_idx, wait_idx, issue_gather, wait_gather, compute,
              issue_scatter, wait_scatter)
    plsc.subcore_barrier()
    pltpu.sync_copy(accum.at[pl.ds(s * RPT, RPT)],
                    out.at[c, pl.ds(s * RPT, RPT)])


_edges1 = functools.partial(
    pl.kernel,
    out_type=jax.ShapeDtypeStruct((NC, NPAD, ROW1), _f32),
    mesh=_MESH,
    compiler_params=_SC_PARAMS,
    scratch_types=[
        pltpu.VMEM_SHARED((NPAD, ROW1), _f32),
        pltpu.VMEM((B, ROW1), _f32),
        pltpu.VMEM((B, ROW1), _f32),
        pltpu.VMEM((B, HEADS), _f32),
        pltpu.VMEM((B, HEADS), _f32),
        pltpu.VMEM((B * HEADS,), _f32),
        pltpu.VMEM((16,), _f32),
        pltpu.VMEM((B,), jnp.int32),
        pltpu.VMEM((B,), jnp.int32),
        pltpu.VMEM((B,), jnp.int32),
        pltpu.VMEM((B,), jnp.int32),
        pltpu.VMEM((B,), jnp.int32),
        pltpu.VMEM((B,), jnp.int32),
        pltpu.VMEM((B,), _f32),
        pltpu.VMEM((B,), _f32),
        pltpu.VMEM((B,), _f32),
        pltpu.SemaphoreType.DMA,
        pltpu.SemaphoreType.DMA,
        pltpu.SemaphoreType.DMA,
        pltpu.SemaphoreType.DMA,
        pltpu.SemaphoreType.DMA,
        pltpu.SemaphoreType.DMA,
        pltpu.SemaphoreType.DMA,
    ],
)(_edges1_body)


def _edges2_body(tab2, ad2, src, dst, ea, c16, zeros, out, accum, adt,
                 rows0, rows1, exb, c16b,
                 sb0, sb1, sb2, db0, db1, db2, eb0, eb1, eb2,
                 sg0, sg1, si0, si1, si2, ss0, ss1):
    rows = [rows0, rows1]
    srcb, dstb, eab = [sb0, sb1, sb2], [db0, db1, db2], [eb0, eb1, eb2]
    semg, semi, sems = [sg0, sg1], [si0, si1, si2], [ss0, ss1]

    c = lax.axis_index("c")
    s = lax.axis_index("s")
    wid = c * NS + s
    pltpu.sync_copy(zeros, accum.at[pl.ds(s * RPT, RPT)])
    pltpu.sync_copy(ad2, adt)
    pltpu.sync_copy(c16, c16b)
    plsc.subcore_barrier()

    iota = lax.iota(jnp.int32, 16)
    zero16 = jnp.zeros((16,), jnp.int32)
    c16v = c16b[...]
    col40 = zero16 + NCLASS
    pick_c = [zero16 + i for i in range(16)]

    def issue_idx(j, t):
        base = wid * EPW + j * B
        pltpu.async_copy(src.at[pl.ds(base, B)], srcb[t], semi[t])
        pltpu.async_copy(dst.at[pl.ds(base, B)], dstb[t], semi[t])
        pltpu.async_copy(ea.at[pl.ds(base, B)], eab[t], semi[t])

    def wait_idx(j, t):
        base = wid * EPW + j * B
        pltpu.make_async_copy(src.at[pl.ds(base, B)], srcb[t], semi[t]).wait()
        pltpu.make_async_copy(dst.at[pl.ds(base, B)], dstb[t], semi[t]).wait()
        pltpu.make_async_copy(ea.at[pl.ds(base, B)], eab[t], semi[t]).wait()

    def issue_gather(b, t):
        pltpu.async_copy(tab2.at[srcb[t]], rows[b], semg[b])

    def wait_gather(b, t):
        pltpu.make_async_copy(tab2.at[srcb[t]], rows[b], semg[b]).wait()

    def issue_scatter(b, t):
        pltpu.async_copy(rows[b], accum.at[dstb[t]], sems[b], add=True)

    def wait_scatter(b, t):
        pltpu.make_async_copy(rows[b], accum.at[dstb[t]], sems[b]).wait()

    def compute(b, t):
        rw, dg, eg = rows[b], dstb[t], eab[t]

        def alpha_grp(i, _):
            rowi = 16 * i + iota
            asg = plsc.load_gather(rw, [rowi, col40])
            dstv = dg[pl.ds(16 * i, 16)]
            adv = plsc.load_gather(adt, [dstv])
            eav = eg[pl.ds(16 * i, 16)]
            tt = asg + adv + eav * c16v
            ex = jnp.exp(jnp.maximum(tt, 0.2 * tt))
            exb[pl.ds(16 * i, 16)] = ex
            return 0

        lax.fori_loop(0, B // 16, alpha_grp, 0)

        def msg_grp(g, _):
            e0 = 16 * g
            exv = exb[pl.ds(e0, 16)]
            for i in range(16):
                e = e0 + i
                bv = _vgather(exv, pick_c[i])
                for k in range(ROW2 // 16):
                    rw[e, pl.ds(16 * k, 16)] = rw[e, pl.ds(16 * k, 16)] * bv
            plsc.store_scatter(rw, [e0 + iota, col40], exv)
            return 0

        lax.fori_loop(0, B // 16, msg_grp, 0)

    _pipeline(issue_idx, wait_idx, issue_gather, wait_gather, compute,
              issue_scatter, wait_scatter)
    plsc.subcore_barrier()
    pltpu.sync_copy(accum.at[pl.ds(s * RPT, RPT)],
                    out.at[c, pl.ds(s * RPT, RPT)])


_edges2 = functools.partial(
    pl.kernel,
    out_type=jax.ShapeDtypeStruct((NC, NPAD, ROW2), _f32),
    mesh=_MESH,
    compiler_params=_SC_PARAMS,
    scratch_types=[
        pltpu.VMEM_SHARED((NPAD, ROW2), _f32),
        pltpu.VMEM((N,), _f32),
        pltpu.VMEM((B, ROW2), _f32),
        pltpu.VMEM((B, ROW2), _f32),
        pltpu.VMEM((B,), _f32),
        pltpu.VMEM((16,), _f32),
        pltpu.VMEM((B,), jnp.int32),
        pltpu.VMEM((B,), jnp.int32),
        pltpu.VMEM((B,), jnp.int32),
        pltpu.VMEM((B,), jnp.int32),
        pltpu.VMEM((B,), jnp.int32),
        pltpu.VMEM((B,), jnp.int32),
        pltpu.VMEM((B,), _f32),
        pltpu.VMEM((B,), _f32),
        pltpu.VMEM((B,), _f32),
        pltpu.SemaphoreType.DMA,
        pltpu.SemaphoreType.DMA,
        pltpu.SemaphoreType.DMA,
        pltpu.SemaphoreType.DMA,
        pltpu.SemaphoreType.DMA,
        pltpu.SemaphoreType.DMA,
        pltpu.SemaphoreType.DMA,
    ],
)(_edges2_body)


def kernel(x, edge_index, edge_attr, W1, att_src1, att_dst1, We1, att_e1, b1,
           W2, att_src2, att_dst2, We2, att_e2, b2):
    src = edge_index[0]
    dst = edge_index[1]
    ea = edge_attr[:, 0]

    eye8 = jnp.repeat(jnp.eye(HEADS, dtype=_f32), NHID, axis=0)
    Asrc = eye8 * att_src1.reshape(HC, 1)
    Adst = eye8 * att_dst1.reshape(HC, 1)
    exp8 = jnp.repeat(jnp.eye(HEADS, dtype=_f32), NHID, axis=1)
    c1 = (We1.reshape(HEADS, NHID) * att_e1[0]).sum(-1)
    c1_16 = jnp.tile(c1, 2)
    c2 = (We2.reshape(NCLASS) * att_e2[0, 0]).sum()
    c2_16 = jnp.full((16,), c2, _f32)
    zeros1 = jnp.zeros((RPT, ROW1), _f32)
    zeros2 = jnp.zeros((RPT, ROW2), _f32)

    eam = _ea_mean(ea)
    tab, ad = _prep1(x, W1, Asrc, Adst)
    acc1 = _edges1(tab, ad, src, dst, ea, c1_16, zeros1)
    tab2, ad2 = _mid(tab, acc1, ad, eam, c1.reshape(1, HEADS),
                     b1.reshape(1, HC), exp8, W2,
                     att_src2.reshape(NCLASS, 1), att_dst2.reshape(NCLASS, 1))
    acc2 = _edges2(tab2, ad2.reshape(N), src, dst, ea, c2_16, zeros2)
    return _final(tab2, acc2, ad2, eam, c2.reshape(1, 1), b2.reshape(1, NCLASS))
